# baseline (speedup 1.0000x reference)
import jax
import jax.numpy as jnp
from jax.experimental import pallas as pl
from jax.experimental.pallas import tpu as pltpu

_H = 16
_CHUNK = 128


def _mm_kernel(x_ref, w_ref, o_ref):
    o_ref[...] = jnp.dot(x_ref[...], w_ref[...],
                         preferred_element_type=jnp.float32)


def _mm(x2, w, bm=512):
    m, k = x2.shape
    n = w.shape[1]
    return pl.pallas_call(
        _mm_kernel,
        grid=(m // bm,),
        in_specs=[
            pl.BlockSpec((bm, k), lambda i: (i, 0)),
            pl.BlockSpec((k, n), lambda i: (0, 0)),
        ],
        out_specs=pl.BlockSpec((bm, n), lambda i: (i, 0)),
        out_shape=jax.ShapeDtypeStruct((m, n), jnp.float32),
        compiler_params=pltpu.CompilerParams(
            dimension_semantics=("parallel",),
            vmem_limit_bytes=60 * 1024 * 1024,
        ),
        name="qkv_proj",
    )(x2, w)


def _mm_bias_kernel(x_ref, w_ref, b_ref, o_ref):
    o_ref[...] = jnp.dot(x_ref[...], w_ref[...],
                         preferred_element_type=jnp.float32) + b_ref[...]


def _mm_bias(x2, w, b2, bm=512):
    m, k = x2.shape
    n = w.shape[1]
    return pl.pallas_call(
        _mm_bias_kernel,
        grid=(m // bm,),
        in_specs=[
            pl.BlockSpec((bm, k), lambda i: (i, 0)),
            pl.BlockSpec((k, n), lambda i: (0, 0)),
            pl.BlockSpec((1, n), lambda i: (0, 0)),
        ],
        out_specs=pl.BlockSpec((bm, n), lambda i: (i, 0)),
        out_shape=jax.ShapeDtypeStruct((m, n), jnp.float32),
        compiler_params=pltpu.CompilerParams(
            dimension_semantics=("parallel",),
            vmem_limit_bytes=60 * 1024 * 1024,
        ),
        name="out_proj",
    )(x2, w, b2)


def _scan_kernel(scal_ref, q_ref, k_ref, v_ref, z_ref, sT_ref,
                 S_scr, V_scr, G_scr):
    c = pl.program_id(1)
    nc = pl.num_programs(1)

    eta = scal_ref[0]
    forget = scal_ref[1]
    beta = scal_ref[2]
    s_scale = scal_ref[3]
    cS = 1.0 - forget
    cV = 1.0 - beta
    c1 = eta * s_scale * (1.0 / 64.0)

    @pl.when(c == 0)
    def _():
        S_scr[...] = jnp.zeros_like(S_scr)
        V_scr[...] = jnp.zeros_like(V_scr)
        G_scr[...] = jnp.zeros_like(G_scr)

    li = jax.lax.broadcasted_iota(jnp.int32, (128, 128), 0) % 16
    lj = jax.lax.broadcasted_iota(jnp.int32, (128, 128), 1) % 16
    mbd = (li == lj).astype(jnp.float32)

    def body(t, gp):
        Sm = S_scr[...]
        Vm = V_scr[...]
        kT = k_ref[0, t]
        qT = q_ref[0, t]
        vrow = v_ref[0, t]
        KX = pltpu.repeat(kT, 8, axis=1)
        QX = pltpu.repeat(qT, 8, axis=1)
        p = jnp.sum(Sm * KX[None], axis=1)
        r = jnp.sum(Vm * KX[None], axis=1)
        v_hat = cS * p - gp * r
        e_t = v_hat - vrow
        g1 = jnp.dot(e_t * e_t, mbd, preferred_element_type=jnp.float32)
        g2 = g1 + jnp.roll(g1, 4, axis=0)
        g2 = g2 + jnp.roll(g2, 2, axis=0)
        g2 = g2 + jnp.roll(g2, 1, axis=0)
        g = eta + c1 * g2
        ec = (cV * e_t)[:, None, :]
        Vt = beta * Vm + ec * KX[None]
        gpx = pltpu.repeat(gp, 8, axis=0)
        Sm1 = cS * Sm - gpx[None] * Vm
        V_scr[...] = Vt
        S_scr[...] = Sm1
        pz = jnp.sum(Sm1 * QX[None], axis=1)
        rz = jnp.sum(Vt * QX[None], axis=1)
        z_ref[0, t] = cS * pz - g * rz
        return g

    gp = jax.lax.fori_loop(0, _CHUNK, body, G_scr[...])
    G_scr[...] = gp

    @pl.when(c == nc - 1)
    def _():
        gx = pltpu.repeat(G_scr[...], 8, axis=0)
        sT_ref[0] = cS * S_scr[...] - gx[None] * V_scr[...]


def _scan(qT5, kT5, vx, scal):
    b, l = vx.shape[0], vx.shape[1]
    nc = l // _CHUNK
    tblk = (1, _CHUNK, 64, 16)
    z, s_t = pl.pallas_call(
        _scan_kernel,
        grid=(b, nc),
        in_specs=[
            pl.BlockSpec(memory_space=pltpu.SMEM),
            pl.BlockSpec(tblk, lambda i, j: (i, j, 0, 0)),
            pl.BlockSpec(tblk, lambda i, j: (i, j, 0, 0)),
            pl.BlockSpec((1, _CHUNK, 8, 128), lambda i, j: (i, j, 0, 0)),
        ],
        out_specs=[
            pl.BlockSpec((1, _CHUNK, 8, 128), lambda i, j: (i, j, 0, 0)),
            pl.BlockSpec((1, 8, 64, 128), lambda i, j: (i, 0, 0, 0)),
        ],
        out_shape=[
            jax.ShapeDtypeStruct((b, l, 8, 128), jnp.float32),
            jax.ShapeDtypeStruct((b, 8, 64, 128), jnp.float32),
        ],
        scratch_shapes=[pltpu.VMEM((8, 64, 128), jnp.float32),
                        pltpu.VMEM((8, 64, 128), jnp.float32),
                        pltpu.VMEM((8, 128), jnp.float32)],
        compiler_params=pltpu.CompilerParams(
            dimension_semantics=("parallel", "arbitrary"),
            vmem_limit_bytes=60 * 1024 * 1024,
        ),
        name="delta_rule_scan",
    )(scal, qT5, kT5, vx)
    return z, s_t


def kernel(x, Wq, Wk, Wv, Wo, bo, log_eta, alpha, logit_beta, surprise_scale):
    b, l, d_in = x.shape
    d_out = Wq.shape[1]
    h = _H
    d = d_out // h

    eta = jax.nn.softplus(log_eta)[0]
    forget = jax.nn.sigmoid(alpha)[0]
    beta = jax.nn.sigmoid(logit_beta)[0]
    s_scale = surprise_scale[0]
    scal = jnp.stack([eta, forget, beta, s_scale])

    wqkv = jnp.concatenate([Wq, Wk, Wv], axis=1)
    qkv = _mm(x.reshape(b * l, d_in), wqkv)
    qkv4 = qkv.reshape(b, l, 3, h, d)

    qT5 = qkv4[:, :, 0].transpose(0, 1, 3, 2)
    kT5 = qkv4[:, :, 1].transpose(0, 1, 3, 2)
    vx = (qkv4[:, :, 2].reshape(b, l, h, 8, 8)
          .transpose(0, 1, 3, 4, 2).reshape(b, l, 8, 128))

    z, s_flat = _scan(qT5, kT5, vx, scal)

    z2 = (z.reshape(b, l, 8, 8, h).transpose(0, 1, 4, 2, 3)
          .reshape(b * l, d_out))
    s_t = (s_flat.reshape(b, 8, d, 8, h).transpose(0, 4, 1, 3, 2)
           .reshape(b, h, d, d))

    out = _mm_bias(z2, Wo, bo.reshape(1, d_out))
    return out.reshape(b, l, d_out), s_t


# speedup vs baseline: 1.2559x; 1.2559x over previous
import jax
import jax.numpy as jnp
from jax.experimental import pallas as pl
from jax.experimental.pallas import tpu as pltpu

_H = 16
_CHUNK = 128


def _mm_kernel(x_ref, w_ref, o_ref):
    o_ref[...] = jnp.dot(x_ref[...], w_ref[...],
                         preferred_element_type=jnp.float32)


def _mm(x2, w, bm=512):
    m, k = x2.shape
    n = w.shape[1]
    return pl.pallas_call(
        _mm_kernel,
        grid=(m // bm,),
        in_specs=[
            pl.BlockSpec((bm, k), lambda i: (i, 0)),
            pl.BlockSpec((k, n), lambda i: (0, 0)),
        ],
        out_specs=pl.BlockSpec((bm, n), lambda i: (i, 0)),
        out_shape=jax.ShapeDtypeStruct((m, n), jnp.float32),
        compiler_params=pltpu.CompilerParams(
            dimension_semantics=("parallel",),
            vmem_limit_bytes=60 * 1024 * 1024,
        ),
        name="qkv_proj",
    )(x2, w)


def _mm_bias_kernel(x_ref, w_ref, b_ref, o_ref):
    o_ref[...] = jnp.dot(x_ref[...], w_ref[...],
                         preferred_element_type=jnp.float32) + b_ref[...]


def _mm_bias(x2, w, b2, bm=512):
    m, k = x2.shape
    n = w.shape[1]
    return pl.pallas_call(
        _mm_bias_kernel,
        grid=(m // bm,),
        in_specs=[
            pl.BlockSpec((bm, k), lambda i: (i, 0)),
            pl.BlockSpec((k, n), lambda i: (0, 0)),
            pl.BlockSpec((1, n), lambda i: (0, 0)),
        ],
        out_specs=pl.BlockSpec((bm, n), lambda i: (i, 0)),
        out_shape=jax.ShapeDtypeStruct((m, n), jnp.float32),
        compiler_params=pltpu.CompilerParams(
            dimension_semantics=("parallel",),
            vmem_limit_bytes=60 * 1024 * 1024,
        ),
        name="out_proj",
    )(x2, w, b2)


def _scan_kernel(scal_ref, q_ref, k_ref, v_ref, z_ref, sT_ref,
                 S_scr, V_scr):
    c = pl.program_id(1)

    eta = scal_ref[0]
    forget = scal_ref[1]
    beta = scal_ref[2]
    s_scale = scal_ref[3]
    cS = 1.0 - forget
    cV = 1.0 - beta
    c1 = eta * s_scale * (1.0 / 64.0)

    @pl.when(c == 0)
    def _():
        S_scr[...] = jnp.zeros_like(S_scr)
        V_scr[...] = jnp.zeros_like(V_scr)

    li = jax.lax.broadcasted_iota(jnp.int32, (128, 128), 0) % 16
    lj = jax.lax.broadcasted_iota(jnp.int32, (128, 128), 1) % 16
    mbd = (li == lj).astype(jnp.float32)

    def step_one(bb, t):
        S = S_scr[bb]
        V = V_scr[bb]
        kT = k_ref[bb, t]
        qT = q_ref[bb, t]
        vrow = v_ref[bb, t]
        KX = pltpu.repeat(kT, 8, axis=1)
        QX = pltpu.repeat(qT, 8, axis=1)
        v_hat = jnp.sum(S * KX[None], axis=1)
        e_t = v_hat - vrow
        g1 = jnp.dot(e_t * e_t, mbd, preferred_element_type=jnp.float32)
        g2 = g1 + jnp.roll(g1, 4, axis=0)
        g2 = g2 + jnp.roll(g2, 2, axis=0)
        g2 = g2 + jnp.roll(g2, 1, axis=0)
        g = eta + c1 * g2
        gx = pltpu.repeat(g, 8, axis=0)
        ec = (cV * e_t)[:, None, :]
        V_new = beta * V + ec * KX[None]
        S_new = cS * S - gx[None] * V_new
        V_scr[bb] = V_new
        S_scr[bb] = S_new
        z_ref[bb, t] = jnp.sum(S_new * QX[None], axis=1)

    def body(t, _):
        step_one(0, t)
        step_one(1, t)
        return 0

    jax.lax.fori_loop(0, _CHUNK, body, 0)
    sT_ref[...] = S_scr[...]


def _scan(qT5, kT5, vx, scal):
    b, l = vx.shape[0], vx.shape[1]
    nc = l // _CHUNK
    tblk = (2, _CHUNK, 64, 16)
    z, s_t = pl.pallas_call(
        _scan_kernel,
        grid=(b // 2, nc),
        in_specs=[
            pl.BlockSpec(memory_space=pltpu.SMEM),
            pl.BlockSpec(tblk, lambda i, j: (i, j, 0, 0)),
            pl.BlockSpec(tblk, lambda i, j: (i, j, 0, 0)),
            pl.BlockSpec((2, _CHUNK, 8, 128), lambda i, j: (i, j, 0, 0)),
        ],
        out_specs=[
            pl.BlockSpec((2, _CHUNK, 8, 128), lambda i, j: (i, j, 0, 0)),
            pl.BlockSpec((2, 8, 64, 128), lambda i, j: (i, 0, 0, 0)),
        ],
        out_shape=[
            jax.ShapeDtypeStruct((b, l, 8, 128), jnp.float32),
            jax.ShapeDtypeStruct((b, 8, 64, 128), jnp.float32),
        ],
        scratch_shapes=[pltpu.VMEM((2, 8, 64, 128), jnp.float32),
                        pltpu.VMEM((2, 8, 64, 128), jnp.float32)],
        compiler_params=pltpu.CompilerParams(
            dimension_semantics=("parallel", "arbitrary"),
            vmem_limit_bytes=60 * 1024 * 1024,
        ),
        name="delta_rule_scan",
    )(scal, qT5, kT5, vx)
    return z, s_t


def kernel(x, Wq, Wk, Wv, Wo, bo, log_eta, alpha, logit_beta, surprise_scale):
    b, l, d_in = x.shape
    d_out = Wq.shape[1]
    h = _H
    d = d_out // h

    eta = jax.nn.softplus(log_eta)[0]
    forget = jax.nn.sigmoid(alpha)[0]
    beta = jax.nn.sigmoid(logit_beta)[0]
    s_scale = surprise_scale[0]
    scal = jnp.stack([eta, forget, beta, s_scale])

    wqkv = jnp.concatenate([Wq, Wk, Wv], axis=1)
    qkv = _mm(x.reshape(b * l, d_in), wqkv)
    qkv4 = qkv.reshape(b, l, 3, h, d)

    qT5 = qkv4[:, :, 0].transpose(0, 1, 3, 2)
    kT5 = qkv4[:, :, 1].transpose(0, 1, 3, 2)
    vx = (qkv4[:, :, 2].reshape(b, l, h, 8, 8)
          .transpose(0, 1, 3, 4, 2).reshape(b, l, 8, 128))

    z, s_flat = _scan(qT5, kT5, vx, scal)

    z2 = (z.reshape(b, l, 8, 8, h).transpose(0, 1, 4, 2, 3)
          .reshape(b * l, d_out))
    s_t = (s_flat.reshape(b, 8, d, 8, h).transpose(0, 4, 1, 3, 2)
           .reshape(b, h, d, d))

    out = _mm_bias(z2, Wo, bo.reshape(1, d_out))
    return out.reshape(b, l, d_out), s_t


# speedup vs baseline: 1.3347x; 1.0628x over previous
import jax
import jax.numpy as jnp
from jax.experimental import pallas as pl
from jax.experimental.pallas import tpu as pltpu

_H = 16
_CHUNK = 64


def _mm_kernel(x_ref, w_ref, o_ref):
    o_ref[...] = jnp.dot(x_ref[...], w_ref[...],
                         preferred_element_type=jnp.float32)


def _mm(x2, w, bm=512):
    m, k = x2.shape
    n = w.shape[1]
    return pl.pallas_call(
        _mm_kernel,
        grid=(m // bm,),
        in_specs=[
            pl.BlockSpec((bm, k), lambda i: (i, 0)),
            pl.BlockSpec((k, n), lambda i: (0, 0)),
        ],
        out_specs=pl.BlockSpec((bm, n), lambda i: (i, 0)),
        out_shape=jax.ShapeDtypeStruct((m, n), jnp.float32),
        compiler_params=pltpu.CompilerParams(
            dimension_semantics=("parallel",),
            vmem_limit_bytes=60 * 1024 * 1024,
        ),
        name="qkv_proj",
    )(x2, w)


def _mm_bias_kernel(x_ref, w_ref, b_ref, o_ref):
    o_ref[...] = jnp.dot(x_ref[...], w_ref[...],
                         preferred_element_type=jnp.float32) + b_ref[...]


def _mm_bias(x2, w, b2, bm=512):
    m, k = x2.shape
    n = w.shape[1]
    return pl.pallas_call(
        _mm_bias_kernel,
        grid=(m // bm,),
        in_specs=[
            pl.BlockSpec((bm, k), lambda i: (i, 0)),
            pl.BlockSpec((k, n), lambda i: (0, 0)),
            pl.BlockSpec((1, n), lambda i: (0, 0)),
        ],
        out_specs=pl.BlockSpec((bm, n), lambda i: (i, 0)),
        out_shape=jax.ShapeDtypeStruct((m, n), jnp.float32),
        compiler_params=pltpu.CompilerParams(
            dimension_semantics=("parallel",),
            vmem_limit_bytes=60 * 1024 * 1024,
        ),
        name="out_proj",
    )(x2, w, b2)


def _scan_kernel(scal_ref, q_ref, k_ref, v_ref, z_ref, sT_ref,
                 S_scr, V_scr):
    c = pl.program_id(1)

    eta = scal_ref[0]
    forget = scal_ref[1]
    beta = scal_ref[2]
    s_scale = scal_ref[3]
    cS = 1.0 - forget
    cV = 1.0 - beta
    c1 = eta * s_scale * (1.0 / 64.0)

    @pl.when(c == 0)
    def _():
        S_scr[...] = jnp.zeros_like(S_scr)
        V_scr[...] = jnp.zeros_like(V_scr)

    li = jax.lax.broadcasted_iota(jnp.int32, (128, 128), 0) % 16
    lj = jax.lax.broadcasted_iota(jnp.int32, (128, 128), 1) % 16
    mbd = (li == lj).astype(jnp.float32)

    def step_one(bb, t):
        S = S_scr[bb]
        V = V_scr[bb]
        kT = k_ref[bb, t]
        qT = q_ref[bb, t]
        vrow = v_ref[bb, t]
        KX = pltpu.repeat(kT, 8, axis=1)
        QX = pltpu.repeat(qT, 8, axis=1)
        v_hat = jnp.sum(S * KX[None], axis=1)
        e_t = v_hat - vrow
        g1 = jnp.dot(e_t * e_t, mbd, preferred_element_type=jnp.float32)
        g2 = g1 + jnp.roll(g1, 4, axis=0)
        g2 = g2 + jnp.roll(g2, 2, axis=0)
        g2 = g2 + jnp.roll(g2, 1, axis=0)
        g = eta + c1 * g2
        gx = pltpu.repeat(g, 8, axis=0)
        ec = (cV * e_t)[:, None, :]
        V_new = beta * V + ec * KX[None]
        S_new = cS * S - gx[None] * V_new
        V_scr[bb] = V_new
        S_scr[bb] = S_new
        z_ref[bb, t] = jnp.sum(S_new * QX[None], axis=1)

    def body(t, _):
        step_one(0, t)
        step_one(1, t)
        step_one(2, t)
        step_one(3, t)
        return 0

    jax.lax.fori_loop(0, _CHUNK, body, 0)
    sT_ref[...] = S_scr[...]


def _scan(qT5, kT5, vx, scal):
    b, l = vx.shape[0], vx.shape[1]
    nc = l // _CHUNK
    tblk = (4, _CHUNK, 64, 16)
    z, s_t = pl.pallas_call(
        _scan_kernel,
        grid=(b // 4, nc),
        in_specs=[
            pl.BlockSpec(memory_space=pltpu.SMEM),
            pl.BlockSpec(tblk, lambda i, j: (i, j, 0, 0)),
            pl.BlockSpec(tblk, lambda i, j: (i, j, 0, 0)),
            pl.BlockSpec((4, _CHUNK, 8, 128), lambda i, j: (i, j, 0, 0)),
        ],
        out_specs=[
            pl.BlockSpec((4, _CHUNK, 8, 128), lambda i, j: (i, j, 0, 0)),
            pl.BlockSpec((4, 8, 64, 128), lambda i, j: (i, 0, 0, 0)),
        ],
        out_shape=[
            jax.ShapeDtypeStruct((b, l, 8, 128), jnp.float32),
            jax.ShapeDtypeStruct((b, 8, 64, 128), jnp.float32),
        ],
        scratch_shapes=[pltpu.VMEM((4, 8, 64, 128), jnp.float32),
                        pltpu.VMEM((4, 8, 64, 128), jnp.float32)],
        compiler_params=pltpu.CompilerParams(
            dimension_semantics=("parallel", "arbitrary"),
            vmem_limit_bytes=60 * 1024 * 1024,
        ),
        name="delta_rule_scan",
    )(scal, qT5, kT5, vx)
    return z, s_t


def kernel(x, Wq, Wk, Wv, Wo, bo, log_eta, alpha, logit_beta, surprise_scale):
    b, l, d_in = x.shape
    d_out = Wq.shape[1]
    h = _H
    d = d_out // h

    eta = jax.nn.softplus(log_eta)[0]
    forget = jax.nn.sigmoid(alpha)[0]
    beta = jax.nn.sigmoid(logit_beta)[0]
    s_scale = surprise_scale[0]
    scal = jnp.stack([eta, forget, beta, s_scale])

    wqkv = jnp.concatenate([Wq, Wk, Wv], axis=1)
    qkv = _mm(x.reshape(b * l, d_in), wqkv)
    qkv4 = qkv.reshape(b, l, 3, h, d)

    qT5 = qkv4[:, :, 0].transpose(0, 1, 3, 2)
    kT5 = qkv4[:, :, 1].transpose(0, 1, 3, 2)
    vx = (qkv4[:, :, 2].reshape(b, l, h, 8, 8)
          .transpose(0, 1, 3, 4, 2).reshape(b, l, 8, 128))

    z, s_flat = _scan(qT5, kT5, vx, scal)

    z2 = (z.reshape(b, l, 8, 8, h).transpose(0, 1, 4, 2, 3)
          .reshape(b * l, d_out))
    s_t = (s_flat.reshape(b, 8, d, 8, h).transpose(0, 4, 1, 3, 2)
           .reshape(b, h, d, d))

    out = _mm_bias(z2, Wo, bo.reshape(1, d_out))
    return out.reshape(b, l, d_out), s_t


# speedup vs baseline: 1.5441x; 1.1568x over previous
import jax
import jax.numpy as jnp
from jax.experimental import pallas as pl
from jax.experimental.pallas import tpu as pltpu

_H = 16
_CHUNK = 64


def _mm3_kernel(x_ref, wq_ref, wk_ref, wv_ref, oq_ref, ok_ref, ov_ref):
    x = x_ref[...]
    oq_ref[...] = jnp.dot(x, wq_ref[...], preferred_element_type=jnp.float32)
    ok_ref[...] = jnp.dot(x, wk_ref[...], preferred_element_type=jnp.float32)
    ov_ref[...] = jnp.dot(x, wv_ref[...], preferred_element_type=jnp.float32)


def _mm3(x2, wq, wk, wv, bm=512):
    m, k = x2.shape
    n = wq.shape[1]
    wspec = pl.BlockSpec((k, n), lambda i: (0, 0))
    ospec = pl.BlockSpec((bm, n), lambda i: (i, 0))
    osh = jax.ShapeDtypeStruct((m, n), jnp.float32)
    return pl.pallas_call(
        _mm3_kernel,
        grid=(m // bm,),
        in_specs=[pl.BlockSpec((bm, k), lambda i: (i, 0)), wspec, wspec, wspec],
        out_specs=[ospec, ospec, ospec],
        out_shape=[osh, osh, osh],
        compiler_params=pltpu.CompilerParams(
            dimension_semantics=("parallel",),
            vmem_limit_bytes=60 * 1024 * 1024,
        ),
        name="qkv_proj",
    )(x2, wq, wk, wv)


def _mm_bias_kernel(x_ref, w_ref, b_ref, o_ref):
    o_ref[...] = jnp.dot(x_ref[...], w_ref[...],
                         preferred_element_type=jnp.float32) + b_ref[...]


def _mm_bias(x2, w, b2, bm=512):
    m, k = x2.shape
    n = w.shape[1]
    return pl.pallas_call(
        _mm_bias_kernel,
        grid=(m // bm,),
        in_specs=[
            pl.BlockSpec((bm, k), lambda i: (i, 0)),
            pl.BlockSpec((k, n), lambda i: (0, 0)),
            pl.BlockSpec((1, n), lambda i: (0, 0)),
        ],
        out_specs=pl.BlockSpec((bm, n), lambda i: (i, 0)),
        out_shape=jax.ShapeDtypeStruct((m, n), jnp.float32),
        compiler_params=pltpu.CompilerParams(
            dimension_semantics=("parallel",),
            vmem_limit_bytes=60 * 1024 * 1024,
        ),
        name="out_proj",
    )(x2, w, b2)


def _scan_kernel(scal_ref, q_ref, k_ref, v_ref, z_ref, sT_ref,
                 S_scr, V_scr):
    c = pl.program_id(1)

    eta = scal_ref[0]
    forget = scal_ref[1]
    beta = scal_ref[2]
    s_scale = scal_ref[3]
    cS = 1.0 - forget
    cV = 1.0 - beta
    c1 = eta * s_scale * (1.0 / 64.0)

    @pl.when(c == 0)
    def _():
        S_scr[...] = jnp.zeros_like(S_scr)
        V_scr[...] = jnp.zeros_like(V_scr)

    li = jax.lax.broadcasted_iota(jnp.int32, (128, 128), 0) % 16
    lj = jax.lax.broadcasted_iota(jnp.int32, (128, 128), 1) % 16
    mbd = (li == lj).astype(jnp.float32)

    def step_one(bb, t):
        S = S_scr[bb]
        V = V_scr[bb]
        kT = k_ref[bb, t]
        qT = q_ref[bb, t]
        vrow = v_ref[bb, t]
        KX = pltpu.repeat(kT, 8, axis=1)
        QX = pltpu.repeat(qT, 8, axis=1)
        v_hat = jnp.sum(S * KX[None], axis=1)
        e_t = v_hat - vrow
        g1 = jnp.dot(e_t * e_t, mbd, preferred_element_type=jnp.float32)
        g2 = g1 + jnp.roll(g1, 4, axis=0)
        g2 = g2 + jnp.roll(g2, 2, axis=0)
        g2 = g2 + jnp.roll(g2, 1, axis=0)
        g = eta + c1 * g2
        gx = pltpu.repeat(g, 8, axis=0)
        ec = (cV * e_t)[:, None, :]
        V_new = beta * V + ec * KX[None]
        S_new = cS * S - gx[None] * V_new
        V_scr[bb] = V_new
        S_scr[bb] = S_new
        z_ref[bb, t] = jnp.sum(S_new * QX[None], axis=1)

    def body(t, _):
        step_one(0, t)
        step_one(1, t)
        step_one(2, t)
        step_one(3, t)
        return 0

    jax.lax.fori_loop(0, _CHUNK, body, 0)
    sT_ref[...] = S_scr[...]


def _scan(qT5, kT5, vx, scal):
    b, l = vx.shape[0], vx.shape[1]
    nc = l // _CHUNK
    tblk = (4, _CHUNK, 64, 16)
    z, s_t = pl.pallas_call(
        _scan_kernel,
        grid=(b // 4, nc),
        in_specs=[
            pl.BlockSpec(memory_space=pltpu.SMEM),
            pl.BlockSpec(tblk, lambda i, j: (i, j, 0, 0)),
            pl.BlockSpec(tblk, lambda i, j: (i, j, 0, 0)),
            pl.BlockSpec((4, _CHUNK, 8, 128), lambda i, j: (i, j, 0, 0)),
        ],
        out_specs=[
            pl.BlockSpec((4, _CHUNK, 8, 128), lambda i, j: (i, j, 0, 0)),
            pl.BlockSpec((4, 8, 64, 128), lambda i, j: (i, 0, 0, 0)),
        ],
        out_shape=[
            jax.ShapeDtypeStruct((b, l, 8, 128), jnp.float32),
            jax.ShapeDtypeStruct((b, 8, 64, 128), jnp.float32),
        ],
        scratch_shapes=[pltpu.VMEM((4, 8, 64, 128), jnp.float32),
                        pltpu.VMEM((4, 8, 64, 128), jnp.float32)],
        compiler_params=pltpu.CompilerParams(
            dimension_semantics=("parallel", "arbitrary"),
            vmem_limit_bytes=60 * 1024 * 1024,
        ),
        name="delta_rule_scan",
    )(scal, qT5, kT5, vx)
    return z, s_t


def kernel(x, Wq, Wk, Wv, Wo, bo, log_eta, alpha, logit_beta, surprise_scale):
    b, l, d_in = x.shape
    d_out = Wq.shape[1]
    h = _H
    d = d_out // h

    eta = jax.nn.softplus(log_eta)[0]
    forget = jax.nn.sigmoid(alpha)[0]
    beta = jax.nn.sigmoid(logit_beta)[0]
    s_scale = surprise_scale[0]
    scal = jnp.stack([eta, forget, beta, s_scale])

    wq_p = Wq.reshape(d_in, h, d).transpose(0, 2, 1).reshape(d_in, d_out)
    wk_p = Wk.reshape(d_in, h, d).transpose(0, 2, 1).reshape(d_in, d_out)
    wv_p = (Wv.reshape(d_in, h, 8, 8).transpose(0, 2, 3, 1)
            .reshape(d_in, d_out))
    wo_p = Wo.reshape(h, 8, 8, d_out).transpose(1, 2, 0, 3).reshape(d_out, d_out)

    rq, rk, rv = _mm3(x.reshape(b * l, d_in), wq_p, wk_p, wv_p)
    qT5 = rq.reshape(b, l, d, h)
    kT5 = rk.reshape(b, l, d, h)
    vx = rv.reshape(b, l, 8, 128)

    z, s_flat = _scan(qT5, kT5, vx, scal)

    s_t = (s_flat.reshape(b, 8, d, 8, h).transpose(0, 4, 1, 3, 2)
           .reshape(b, h, d, d))

    out = _mm_bias(z.reshape(b * l, d_out), wo_p, bo.reshape(1, d_out))
    return out.reshape(b, l, d_out), s_t


# speedup vs baseline: 1.6193x; 1.0487x over previous
import jax
import jax.numpy as jnp
from jax.experimental import pallas as pl
from jax.experimental.pallas import tpu as pltpu

_H = 16
_CHUNK = 256


def _mm3_kernel(x_ref, wq_ref, wk_ref, wv_ref, oq_ref, ok_ref, ov_ref):
    x = x_ref[...]
    oq_ref[...] = jnp.dot(x, wq_ref[...], preferred_element_type=jnp.float32)
    ok_ref[...] = jnp.dot(x, wk_ref[...], preferred_element_type=jnp.float32)
    ov_ref[...] = jnp.dot(x, wv_ref[...], preferred_element_type=jnp.float32)


def _mm3(x2, wq, wk, wv, bm=512):
    m, k = x2.shape
    n = wq.shape[1]
    wspec = pl.BlockSpec((k, n), lambda i: (0, 0))
    ospec = pl.BlockSpec((bm, n), lambda i: (i, 0))
    osh = jax.ShapeDtypeStruct((m, n), jnp.float32)
    return pl.pallas_call(
        _mm3_kernel,
        grid=(m // bm,),
        in_specs=[pl.BlockSpec((bm, k), lambda i: (i, 0)), wspec, wspec, wspec],
        out_specs=[ospec, ospec, ospec],
        out_shape=[osh, osh, osh],
        compiler_params=pltpu.CompilerParams(
            dimension_semantics=("parallel",),
            vmem_limit_bytes=60 * 1024 * 1024,
        ),
        name="qkv_proj",
    )(x2, wq, wk, wv)


def _mm_bias_kernel(x_ref, w_ref, b_ref, o_ref):
    o_ref[...] = jnp.dot(x_ref[...], w_ref[...],
                         preferred_element_type=jnp.float32) + b_ref[...]


def _mm_bias(x2, w, b2, bm=512):
    m, k = x2.shape
    n = w.shape[1]
    return pl.pallas_call(
        _mm_bias_kernel,
        grid=(m // bm,),
        in_specs=[
            pl.BlockSpec((bm, k), lambda i: (i, 0)),
            pl.BlockSpec((k, n), lambda i: (0, 0)),
            pl.BlockSpec((1, n), lambda i: (0, 0)),
        ],
        out_specs=pl.BlockSpec((bm, n), lambda i: (i, 0)),
        out_shape=jax.ShapeDtypeStruct((m, n), jnp.float32),
        compiler_params=pltpu.CompilerParams(
            dimension_semantics=("parallel",),
            vmem_limit_bytes=60 * 1024 * 1024,
        ),
        name="out_proj",
    )(x2, w, b2)


def _scan_kernel(scal_ref, q_ref, k_ref, v_ref, z_ref, sT_ref,
                 S_scr, V_scr):
    c = pl.program_id(1)

    eta = scal_ref[0]
    forget = scal_ref[1]
    beta = scal_ref[2]
    s_scale = scal_ref[3]
    cS = 1.0 - forget
    cV = 1.0 - beta
    c1 = eta * s_scale * (1.0 / 64.0)

    @pl.when(c == 0)
    def _():
        S_scr[...] = jnp.zeros_like(S_scr)
        V_scr[...] = jnp.zeros_like(V_scr)

    li = jax.lax.broadcasted_iota(jnp.int32, (128, 128), 0) % 16
    lj = jax.lax.broadcasted_iota(jnp.int32, (128, 128), 1) % 16
    mbd = (li == lj).astype(jnp.float32)

    def step_one(bb, t):
        S = S_scr[bb]
        V = V_scr[bb]
        kT = k_ref[bb, t].T
        qT = q_ref[bb, t].T
        vrow = v_ref[bb, t]
        KX = pltpu.repeat(kT, 8, axis=1)
        QX = pltpu.repeat(qT, 8, axis=1)
        v_hat = jnp.sum(S * KX[None], axis=1)
        e_t = v_hat - vrow
        g1 = jnp.dot(e_t * e_t, mbd, preferred_element_type=jnp.float32)
        g2 = g1 + jnp.roll(g1, 4, axis=0)
        g2 = g2 + jnp.roll(g2, 2, axis=0)
        g2 = g2 + jnp.roll(g2, 1, axis=0)
        g = eta + c1 * g2
        gx = pltpu.repeat(g, 8, axis=0)
        ec = (cV * e_t)[:, None, :]
        V_new = beta * V + ec * KX[None]
        S_new = cS * S - gx[None] * V_new
        V_scr[bb] = V_new
        S_scr[bb] = S_new
        z_ref[bb, t] = jnp.sum(S_new * QX[None], axis=1)

    def body(t, _):
        step_one(0, t)
        step_one(1, t)
        step_one(2, t)
        step_one(3, t)
        return 0

    jax.lax.fori_loop(0, _CHUNK, body, 0)
    sT_ref[...] = S_scr[...]


def _scan(qT5, kT5, vx, scal):
    b, l = vx.shape[0], vx.shape[1]
    nc = l // _CHUNK
    tblk = (4, _CHUNK, 16, 64)
    z, s_t = pl.pallas_call(
        _scan_kernel,
        grid=(b // 4, nc),
        in_specs=[
            pl.BlockSpec(memory_space=pltpu.SMEM),
            pl.BlockSpec(tblk, lambda i, j: (i, j, 0, 0)),
            pl.BlockSpec(tblk, lambda i, j: (i, j, 0, 0)),
            pl.BlockSpec((4, _CHUNK, 8, 128), lambda i, j: (i, j, 0, 0)),
        ],
        out_specs=[
            pl.BlockSpec((4, _CHUNK, 8, 128), lambda i, j: (i, j, 0, 0)),
            pl.BlockSpec((4, 8, 64, 128), lambda i, j: (i, 0, 0, 0)),
        ],
        out_shape=[
            jax.ShapeDtypeStruct((b, l, 8, 128), jnp.float32),
            jax.ShapeDtypeStruct((b, 8, 64, 128), jnp.float32),
        ],
        scratch_shapes=[pltpu.VMEM((4, 8, 64, 128), jnp.float32),
                        pltpu.VMEM((4, 8, 64, 128), jnp.float32)],
        compiler_params=pltpu.CompilerParams(
            dimension_semantics=("parallel", "arbitrary"),
            vmem_limit_bytes=60 * 1024 * 1024,
        ),
        name="delta_rule_scan",
    )(scal, qT5, kT5, vx)
    return z, s_t


def kernel(x, Wq, Wk, Wv, Wo, bo, log_eta, alpha, logit_beta, surprise_scale):
    b, l, d_in = x.shape
    d_out = Wq.shape[1]
    h = _H
    d = d_out // h

    eta = jax.nn.softplus(log_eta)[0]
    forget = jax.nn.sigmoid(alpha)[0]
    beta = jax.nn.sigmoid(logit_beta)[0]
    s_scale = surprise_scale[0]
    scal = jnp.stack([eta, forget, beta, s_scale])

    wq_p = Wq
    wk_p = Wk
    wv_p = (Wv.reshape(d_in, h, 8, 8).transpose(0, 2, 3, 1)
            .reshape(d_in, d_out))
    wo_p = Wo.reshape(h, 8, 8, d_out).transpose(1, 2, 0, 3).reshape(d_out, d_out)

    rq, rk, rv = _mm3(x.reshape(b * l, d_in), wq_p, wk_p, wv_p)
    qT5 = rq.reshape(b, l, h, d)
    kT5 = rk.reshape(b, l, h, d)
    vx = rv.reshape(b, l, 8, 128)

    z, s_flat = _scan(qT5, kT5, vx, scal)

    s_t = (s_flat.reshape(b, 8, d, 8, h).transpose(0, 4, 1, 3, 2)
           .reshape(b, h, d, d))

    out = _mm_bias(z.reshape(b * l, d_out), wo_p, bo.reshape(1, d_out))
    return out.reshape(b, l, d_out), s_t
